# baseline (speedup 1.0000x reference)
import functools

import jax
import jax.numpy as jnp
from jax import lax
from jax.experimental import pallas as pl
from jax.experimental.pallas import tpu as pltpu
from jax.experimental.pallas import tpu_sc as plsc

B = 4096
HIST = 50
D = 32
L = 16
NC = 2
NS = 16
NW = NC * NS
RPW = B // NW
CH = 32
NCHUNK = RPW // CH
IDX_PER_CHUNK = CH * HIST
GW = HIST
NG = IDX_PER_CHUNK // GW
NGRP = (HIST + L - 1) // L


def _body(uidx_hbm, iidx_hbm, utab_hbm, itab_hbm, out_hbm,
          iidx_v, rows_v, uidx_v, urows_v, out_v, sem):
    wid = lax.axis_index("s") * NC + lax.axis_index("c")
    iota = lax.broadcasted_iota(jnp.int32, (L,), 0)

    for c in range(NCHUNK):
        row0 = wid * RPW + c * CH

        pltpu.sync_copy(iidx_hbm.at[pl.ds(row0, NG)], iidx_v)
        pltpu.sync_copy(uidx_hbm.at[pl.ds(row0, CH)], uidx_v)

        user_cp = pltpu.async_copy(utab_hbm.at[uidx_v], urows_v, sem)
        item_cps = [
            pltpu.async_copy(itab_hbm.at[iidx_v.at[k]],
                             rows_v.at[pl.ds(k * GW, GW)], sem)
            for k in range(NG)
        ]
        user_cp.wait()
        for cp in item_cps:
            cp.wait()

        def row_body(r, carry):
            base = r * HIST
            idx0 = [jnp.minimum(base + g * L + iota, IDX_PER_CHUNK - 1)
                    for g in range(NGRP)]
            accs = [jnp.zeros((L,), jnp.float32) for _ in range(NGRP)]
            uvecs = [urows_v[r, pl.ds(h * L, L)] for h in range(D // L)]
            for d in range(D):
                u = uvecs[d // L][d % L]
                col = jnp.full((L,), d, jnp.int32)
                for g in range(NGRP):
                    vals = plsc.load_gather(rows_v, [idx0[g], col])
                    accs[g] = accs[g] + vals * u
            for g in range(NGRP):
                oidx = base + g * L + iota
                mask = (g * L + iota) < HIST
                plsc.store_scatter(out_v, [jnp.minimum(oidx, IDX_PER_CHUNK - 1)],
                                   accs[g], mask=mask)
            return carry

        lax.fori_loop(0, CH, row_body, 0)

        pltpu.sync_copy(out_v, out_hbm.at[pl.ds(row0 * HIST, IDX_PER_CHUNK)])


_sc_call = pl.kernel(
    _body,
    out_type=jax.ShapeDtypeStruct((B * HIST,), jnp.float32),
    mesh=plsc.VectorSubcoreMesh(core_axis_name="c", subcore_axis_name="s"),
    scratch_types=[
        pltpu.VMEM((NG, GW), jnp.int32),
        pltpu.VMEM((IDX_PER_CHUNK, D), jnp.float32),
        pltpu.VMEM((CH,), jnp.int32),
        pltpu.VMEM((CH, D), jnp.float32),
        pltpu.VMEM((IDX_PER_CHUNK,), jnp.float32),
        pltpu.SemaphoreType.DMA,
    ],
    compiler_params=pltpu.CompilerParams(
        needs_layout_passes=False,
        use_tc_tiling_on_sc=False,
    ),
)


def kernel(user_indices, item_indices, user_table, item_table):
    uidx = user_indices.reshape(B).astype(jnp.int32)
    iidx = item_indices.reshape(B * HIST // GW, GW).astype(jnp.int32)
    out = _sc_call(uidx, iidx, user_table, item_table)
    return out.reshape(B, HIST)


# speedup vs baseline: 1.3850x; 1.3850x over previous
import jax
import jax.numpy as jnp
from jax import lax
from jax.experimental import pallas as pl
from jax.experimental.pallas import tpu as pltpu
from jax.experimental.pallas import tpu_sc as plsc

B = 4096
HIST = 50
D = 32
NU = 1000000
L = 16
NC = 2
NS = 16
NW = NC * NS
RPW = B // NW
CH = 8
NCHUNK = RPW // CH
IPC = CH * HIST
GW = 100
NG = IPC // GW
NGRP = (HIST + L - 1) // L
NV = IPC // L
RQ = NU * D // 128


def _body(uidx_hbm, iidx_hbm, utT_hbm, it2_hbm, out_hbm,
          iidx_s, m32_s, idxq_v, rows_v, uidx_s, uw_v, out_v, sem):
    wid = lax.axis_index("s") * NC + lax.axis_index("c")
    iota = lax.broadcasted_iota(jnp.int32, (L,), 0)

    def chunk_body(ch, carry):
        row0 = wid * RPW + ch * CH

        pltpu.sync_copy(iidx_hbm.at[pl.ds(row0, CH)], iidx_s)
        pltpu.sync_copy(uidx_hbm.at[pl.ds(row0, CH)], uidx_s.at[pl.ds(0, CH)])

        for v in range(NV):
            ps = iota + v * L
            rr = ps // HIST
            cc = ps % HIST
            vals = plsc.load_gather(iidx_s, [rr, cc])
            plsc.store_scatter(idxq_v, [ps // GW, ps % GW],
                               jnp.right_shift(vals, 2))
            plsc.store_scatter(m32_s, [rr, cc],
                               jnp.left_shift(jnp.bitwise_and(vals, 3), 5))

        uvec = uidx_s[...]
        ucols = []
        user_cps = []
        for j in range(CH):
            uid = uvec[j]
            c0 = pl.multiple_of(jnp.left_shift(jnp.right_shift(uid, 7), 7), 128)
            ucols.append(jnp.bitwise_and(uid, 127))
            user_cps.append(
                pltpu.async_copy(utT_hbm.at[:, pl.ds(c0, 128)],
                                 uw_v.at[j], sem))
        item_cps = [
            pltpu.async_copy(it2_hbm.at[idxq_v.at[k]],
                             rows_v.at[pl.ds(k * GW, GW)], sem)
            for k in range(NG)
        ]
        for cp in user_cps:
            cp.wait()
        for cp in item_cps:
            cp.wait()

        for r in range(CH):
            jv = jnp.full((L,), r, jnp.int32)
            cv = jnp.full((L,), 1, jnp.int32) * ucols[r]
            u_halves = [
                plsc.load_gather(uw_v, [jv, iota + h * L, cv])
                for h in range(D // L)
            ]
            base = r * HIST
            lclamp = [jnp.minimum(iota + g * L, HIST - 1) for g in range(NGRP)]
            idx0 = [lclamp[g] + base for g in range(NGRP)]
            rr = jnp.full((L,), r, jnp.int32)
            m32 = [plsc.load_gather(m32_s, [rr, lclamp[g]])
                   for g in range(NGRP)]
            accs = [jnp.zeros((L,), jnp.float32) for _ in range(NGRP)]
            for d in range(D):
                u = u_halves[d // L][d % L]
                for g in range(NGRP):
                    vals = plsc.load_gather(rows_v, [idx0[g], m32[g] + d])
                    accs[g] = accs[g] + vals * u
            for g in range(NGRP):
                mask = (g * L + iota) < HIST
                plsc.store_scatter(out_v, [idx0[g]], accs[g], mask=mask)

        pltpu.sync_copy(out_v, out_hbm.at[pl.ds(row0 * HIST, IPC)])
        return carry

    lax.fori_loop(0, NCHUNK, chunk_body, 0)


_sc_call = pl.kernel(
    _body,
    out_type=jax.ShapeDtypeStruct((B * HIST,), jnp.float32),
    mesh=plsc.VectorSubcoreMesh(core_axis_name="c", subcore_axis_name="s"),
    scratch_types=[
        pltpu.VMEM((CH, HIST), jnp.int32),
        pltpu.VMEM((CH, HIST), jnp.int32),
        pltpu.VMEM((NG, GW), jnp.int32),
        pltpu.VMEM((IPC, 128), jnp.float32),
        pltpu.VMEM((L,), jnp.int32),
        pltpu.VMEM((CH, D, 128), jnp.float32),
        pltpu.VMEM((IPC,), jnp.float32),
        pltpu.SemaphoreType.DMA,
    ],
    compiler_params=pltpu.CompilerParams(
        needs_layout_passes=False,
        use_tc_tiling_on_sc=True,
    ),
)


def kernel(user_indices, item_indices, user_table, item_table):
    uidx = user_indices.reshape(B).astype(jnp.int32)
    iidx = item_indices.astype(jnp.int32)
    utT = jnp.swapaxes(user_table, 0, 1)
    it2 = item_table.reshape(RQ, 128)
    out = _sc_call(uidx, iidx, utT, it2)
    return out.reshape(B, HIST)
